# baseline (speedup 1.0000x reference)
import jax
import jax.numpy as jnp
from jax.experimental import pallas as pl
from jax.experimental.pallas import tpu as pltpu

EPS = 1e-09
C = 32


def _fused_kernel(mask_ref, w_ref, wexp_ref, wbias_ref, bias_ref, x_ref,
                  o_ref, xbuf, obuf, insem, outsem):
    j = pl.program_id(1)
    nsteps = pl.num_programs(1)
    b = pl.program_id(0) * nsteps + j
    nf = x_ref.shape[1]
    n = x_ref.shape[2]
    nc = nf // C
    cur = jax.lax.rem(j, 2)
    nxt = jax.lax.rem(j + 1, 2)

    def dma_in(slot, bb, c):
        pltpu.make_async_copy(x_ref.at[bb, pl.ds(c * C, C)],
                              xbuf.at[slot, pl.ds(c * C, C)],
                              insem.at[slot, c]).start()

    def wait_in(slot, c):
        pltpu.make_async_copy(x_ref.at[0, pl.ds(0, C)],
                              xbuf.at[slot, pl.ds(0, C)],
                              insem.at[slot, c]).wait()

    def dma_out(slot, bb, c):
        pltpu.make_async_copy(obuf.at[slot],
                              o_ref.at[bb, pl.ds(c * C, C)],
                              outsem.at[slot]).start()

    def wait_out(slot):
        pltpu.make_async_copy(obuf.at[slot],
                              o_ref.at[0, pl.ds(0, C)],
                              outsem.at[slot]).wait()

    @pl.when(j == 0)
    def _():
        for c in range(nc):
            dma_in(cur, b, c)

    @pl.when(j + 1 < nsteps)
    def _():
        for c in range(nc):
            dma_in(nxt, b + 1, c)

    m = mask_ref[0]
    md = m * m
    lane = jax.lax.broadcasted_iota(jnp.int32, (1, n), 1)
    ii = jax.lax.broadcasted_iota(jnp.int32, (n, n), 0)
    kk = jax.lax.broadcasted_iota(jnp.int32, (n, n), 1)
    eye = ii == kk

    hn = n // 2
    lo = slice(0, hn)
    hi = slice(hn, n)
    e2 = jnp.where(eye, md, 0.0)
    e2a = e2[lo, lo]
    e2b = e2[hi, hi]
    trvec = jnp.zeros((1, n), jnp.float32)
    s2 = jnp.float32(0.0)
    for c in range(nc):
        wait_in(cur, c)
        for g in range(C):
            fidx = c * C + g
            a = xbuf[cur, fidx]
            tr_g = jnp.sum(a[lo, lo] * e2a) + jnp.sum(a[hi, hi] * e2b)
            trvec = trvec + jnp.where(lane == fidx, tr_g, 0.0)
            s2 = s2 + jnp.sum(a * jnp.transpose(a) * md)

    cnt = jnp.sum(md)
    cnt2 = jnp.maximum(cnt - 1.0, 1.0)
    s1 = jnp.sum(trvec)
    s3 = jnp.sum(trvec * trvec)
    mean_b = s1 / (cnt * nf)
    var_b = (s2 - s3 / cnt) / (cnt2 * nf)
    inv = jax.lax.rsqrt(var_b + EPS)

    scale_vec = w_ref[...] * jnp.exp(wexp_ref[...]) + wbias_ref[...]
    flane = jax.lax.broadcasted_iota(jnp.int32, (1, nf), 1)
    e1 = jnp.where(eye, 1.0, 0.0)
    m2i = (jnp.transpose(m) * m) * inv
    me1 = mean_b * e1

    for c in range(nc):
        @pl.when(j > 0)
        def _():
            wait_out(c)
        svals = []
        bvals = []
        for g in range(C):
            fidx = c * C + g
            fsel = flane == fidx
            svals.append(jnp.sum(jnp.where(fsel, scale_vec, 0.0)))
            bvals.append(jnp.sum(jnp.where(fsel, bias_ref[...], 0.0)))
        for rb in range(2):
            rs = lo if rb == 0 else hi
            for cb in range(2):
                cs = lo if cb == 0 else hi
                m2i_q = m2i[rs, cs]
                if cb == rb:
                    me1_q = me1[rs, cs]
                    e1_q = e1[rs, cs]
                    for g in range(C):
                        fidx = c * C + g
                        a_q = xbuf[cur, fidx, rs, cs]
                        obuf[c, g, rs, cs] = (
                            ((a_q - me1_q) * m2i_q) * svals[g]
                            + bvals[g] * e1_q)
                else:
                    for g in range(C):
                        fidx = c * C + g
                        a_q = xbuf[cur, fidx, rs, cs]
                        obuf[c, g, rs, cs] = (a_q * m2i_q) * svals[g]
        dma_out(c, b, c)

    @pl.when(j == nsteps - 1)
    def _():
        for c in range(nc):
            wait_out(c)


def kernel(x, mask, weight, weight_exp, weight_bias, bias):
    b, f, n, _ = x.shape
    half = b // 2
    mask3 = mask.reshape(b, 1, n)
    w2 = weight.reshape(1, f)
    wexp2 = weight_exp.reshape(1, f)
    wb2 = weight_bias.reshape(1, f)
    bias2 = bias.reshape(1, f)

    out = pl.pallas_call(
        _fused_kernel,
        grid=(2, half),
        in_specs=[
            pl.BlockSpec((1, 1, n), lambda i, j: (i * half + j, 0, 0)),
            pl.BlockSpec((1, f), lambda i, j: (0, 0)),
            pl.BlockSpec((1, f), lambda i, j: (0, 0)),
            pl.BlockSpec((1, f), lambda i, j: (0, 0)),
            pl.BlockSpec((1, f), lambda i, j: (0, 0)),
            pl.BlockSpec(memory_space=pl.ANY),
        ],
        out_specs=pl.BlockSpec(memory_space=pl.ANY),
        out_shape=jax.ShapeDtypeStruct((b, f, n, n), jnp.float32),
        scratch_shapes=[
            pltpu.VMEM((2, f, n, n), jnp.float32),
            pltpu.VMEM((f // C, C, n, n), jnp.float32),
            pltpu.SemaphoreType.DMA((2, f // C)),
            pltpu.SemaphoreType.DMA((f // C,)),
        ],
        compiler_params=pltpu.CompilerParams(
            dimension_semantics=("parallel", "arbitrary")),
    )(mask3, w2, wexp2, wb2, bias2, x)
    return out


# speedup vs baseline: 1.0795x; 1.0795x over previous
import jax
import jax.numpy as jnp
from jax.experimental import pallas as pl
from jax.experimental.pallas import tpu as pltpu

EPS = 1e-09
C = 16


def _fused_kernel(mask_ref, w_ref, wexp_ref, wbias_ref, bias_ref, x_ref,
                  o_ref, xbuf, obuf, insem, outsem):
    j = pl.program_id(1)
    nsteps = pl.num_programs(1)
    b = pl.program_id(0) * nsteps + j
    nf = x_ref.shape[1]
    n = x_ref.shape[2]
    nc = nf // C
    cur = jax.lax.rem(j, 2)
    nxt = jax.lax.rem(j + 1, 2)

    def dma_in(slot, bb, c):
        pltpu.make_async_copy(x_ref.at[bb, pl.ds(c * C, C)],
                              xbuf.at[slot, pl.ds(c * C, C)],
                              insem.at[slot, c]).start()

    def wait_in(slot, c):
        pltpu.make_async_copy(x_ref.at[0, pl.ds(0, C)],
                              xbuf.at[slot, pl.ds(0, C)],
                              insem.at[slot, c]).wait()

    def dma_out(slot, bb, c):
        pltpu.make_async_copy(obuf.at[slot],
                              o_ref.at[bb, pl.ds(c * C, C)],
                              outsem.at[slot]).start()

    def wait_out(slot):
        pltpu.make_async_copy(obuf.at[slot],
                              o_ref.at[0, pl.ds(0, C)],
                              outsem.at[slot]).wait()

    @pl.when(j == 0)
    def _():
        for c in range(nc):
            dma_in(cur, b, c)

    @pl.when(j + 1 < nsteps)
    def _():
        for c in range(nc):
            dma_in(nxt, b + 1, c)

    m = mask_ref[0]
    md = m * m
    lane = jax.lax.broadcasted_iota(jnp.int32, (1, n), 1)
    ii = jax.lax.broadcasted_iota(jnp.int32, (n, n), 0)
    kk = jax.lax.broadcasted_iota(jnp.int32, (n, n), 1)
    eye = ii == kk

    hn = n // 2
    lo = slice(0, hn)
    hi = slice(hn, n)
    e2 = jnp.where(eye, md, 0.0)
    e2a = e2[lo, lo]
    e2b = e2[hi, hi]
    trvec = jnp.zeros((1, n), jnp.float32)
    s2 = jnp.float32(0.0)
    for c in range(nc):
        wait_in(cur, c)
        for g in range(C):
            fidx = c * C + g
            a = xbuf[cur, fidx]
            tr_g = jnp.sum(a[lo, lo] * e2a) + jnp.sum(a[hi, hi] * e2b)
            trvec = trvec + jnp.where(lane == fidx, tr_g, 0.0)
            s2 = s2 + jnp.sum(a * jnp.transpose(a) * md)

    cnt = jnp.sum(md)
    cnt2 = jnp.maximum(cnt - 1.0, 1.0)
    s1 = jnp.sum(trvec)
    s3 = jnp.sum(trvec * trvec)
    mean_b = s1 / (cnt * nf)
    var_b = (s2 - s3 / cnt) / (cnt2 * nf)
    inv = jax.lax.rsqrt(var_b + EPS)

    scale_vec = w_ref[...] * jnp.exp(wexp_ref[...]) + wbias_ref[...]
    flane = jax.lax.broadcasted_iota(jnp.int32, (1, nf), 1)
    e1 = jnp.where(eye, 1.0, 0.0)
    m2i = (jnp.transpose(m) * m) * inv
    me1 = mean_b * e1

    for c in range(nc):
        @pl.when(j > 0)
        def _():
            wait_out(c)
        svals = []
        bvals = []
        for g in range(C):
            fidx = c * C + g
            fsel = flane == fidx
            svals.append(jnp.sum(jnp.where(fsel, scale_vec, 0.0)))
            bvals.append(jnp.sum(jnp.where(fsel, bias_ref[...], 0.0)))
        for rb in range(2):
            rs = lo if rb == 0 else hi
            for cb in range(2):
                cs = lo if cb == 0 else hi
                m2i_q = m2i[rs, cs]
                if cb == rb:
                    me1_q = me1[rs, cs]
                    e1_q = e1[rs, cs]
                    for g in range(C):
                        fidx = c * C + g
                        a_q = xbuf[cur, fidx, rs, cs]
                        obuf[c, g, rs, cs] = (
                            ((a_q - me1_q) * m2i_q) * svals[g]
                            + bvals[g] * e1_q)
                else:
                    for g in range(C):
                        fidx = c * C + g
                        a_q = xbuf[cur, fidx, rs, cs]
                        obuf[c, g, rs, cs] = (a_q * m2i_q) * svals[g]
        dma_out(c, b, c)

    @pl.when(j == nsteps - 1)
    def _():
        for c in range(nc):
            wait_out(c)


def kernel(x, mask, weight, weight_exp, weight_bias, bias):
    b, f, n, _ = x.shape
    half = b // 2
    mask3 = mask.reshape(b, 1, n)
    w2 = weight.reshape(1, f)
    wexp2 = weight_exp.reshape(1, f)
    wb2 = weight_bias.reshape(1, f)
    bias2 = bias.reshape(1, f)

    out = pl.pallas_call(
        _fused_kernel,
        grid=(2, half),
        in_specs=[
            pl.BlockSpec((1, 1, n), lambda i, j: (i * half + j, 0, 0)),
            pl.BlockSpec((1, f), lambda i, j: (0, 0)),
            pl.BlockSpec((1, f), lambda i, j: (0, 0)),
            pl.BlockSpec((1, f), lambda i, j: (0, 0)),
            pl.BlockSpec((1, f), lambda i, j: (0, 0)),
            pl.BlockSpec(memory_space=pl.ANY),
        ],
        out_specs=pl.BlockSpec(memory_space=pl.ANY),
        out_shape=jax.ShapeDtypeStruct((b, f, n, n), jnp.float32),
        scratch_shapes=[
            pltpu.VMEM((2, f, n, n), jnp.float32),
            pltpu.VMEM((f // C, C, n, n), jnp.float32),
            pltpu.SemaphoreType.DMA((2, f // C)),
            pltpu.SemaphoreType.DMA((f // C,)),
        ],
        compiler_params=pltpu.CompilerParams(
            dimension_semantics=("parallel", "arbitrary")),
    )(mask3, w2, wexp2, wb2, bias2, x)
    return out
